# Initial kernel scaffold; baseline (speedup 1.0000x reference)
import jax
import jax.numpy as jnp
from jax.experimental import pallas as pl


def kernel(evidence, event_counts, current_step, bank_evidence, bank_step, bank_event_count):
    raise NotImplementedError("write your pallas kernel here")


# speedup vs baseline: 1.6118x; 1.6118x over previous
import jax
import jax.numpy as jnp
from jax.experimental import pallas as pl
from jax.experimental.pallas import tpu as pltpu

NUM_LAYERS = 32
DIM = 2048
TTL = 1024
HALF_LIFE = 256.0
K_CHUNK = 256
NK = TTL // K_CHUNK


def _body(cs_ref, bs_ref, bc_ref, ec_ref, wk_bs_ref, wk_bc_ref,
          ev_ref, bank_ref, out_ref):
    l = pl.program_id(0)
    k = pl.program_id(1)
    cs = cs_ref[0]
    ec = ec_ref[l]

    bs = bs_ref[0]
    bc = bc_ref[0]
    iota = jax.lax.broadcasted_iota(jnp.int32, (1, TTL), 1)
    big = jnp.int32(TTL)
    empty = bs == -1
    first_empty = jnp.min(jnp.where(empty, iota, big))
    minstep = jnp.min(bs)
    oldest = jnp.min(jnp.where(bs == minstep, iota, big))
    slot = jnp.where(first_empty < big, first_empty, oldest)

    valid = (bs >= 0).astype(jnp.float32)
    age = jnp.maximum(cs - bs, 0).astype(jnp.float32)
    w_full = jnp.exp2(-age / HALF_LIFE) * valid * bc
    w_full = jnp.where(iota == slot, 0.0, w_full)
    wsum = jnp.sum(w_full) + ec

    bs_k = wk_bs_ref[0, 0]
    bc_k = wk_bc_ref[0, 0]
    iota_k = jax.lax.broadcasted_iota(jnp.int32, (1, K_CHUNK), 1) + k * K_CHUNK
    valid_k = (bs_k >= 0).astype(jnp.float32)
    age_k = jnp.maximum(cs - bs_k, 0).astype(jnp.float32)
    wk = jnp.exp2(-age_k / HALF_LIFE) * valid_k * bc_k
    wk = jnp.where(iota_k == slot, 0.0, wk)

    partial = jnp.dot(wk, bank_ref[0],
                      preferred_element_type=jnp.float32)

    @pl.when(k == 0)
    def _():
        out_ref[0] = partial

    @pl.when(k > 0)
    def _():
        out_ref[0] = out_ref[0] + partial

    @pl.when(k == NK - 1)
    def _():
        acc = out_ref[0] + ec * ev_ref[0]
        res = acc / jnp.maximum(wsum, 1e-12)
        out_ref[0] = jnp.where(wsum > 0, res, jnp.zeros_like(res))


def kernel(evidence, event_counts, current_step, bank_evidence, bank_step,
           bank_event_count):
    cs = jnp.asarray(current_step, dtype=jnp.int32).reshape(1)
    bs3 = bank_step.reshape(NUM_LAYERS, 1, TTL)
    bc3 = bank_event_count.reshape(NUM_LAYERS, 1, TTL)
    bs_k = bank_step.reshape(NUM_LAYERS, NK, 1, K_CHUNK)
    bc_k = bank_event_count.reshape(NUM_LAYERS, NK, 1, K_CHUNK)
    ev3 = evidence.reshape(NUM_LAYERS, 1, DIM)

    out = pl.pallas_call(
        _body,
        grid=(NUM_LAYERS, NK),
        in_specs=[
            pl.BlockSpec(memory_space=pltpu.SMEM),
            pl.BlockSpec((1, 1, TTL), lambda l, k: (l, 0, 0)),
            pl.BlockSpec((1, 1, TTL), lambda l, k: (l, 0, 0)),
            pl.BlockSpec(memory_space=pltpu.SMEM),
            pl.BlockSpec((1, 1, 1, K_CHUNK), lambda l, k: (l, k, 0, 0)),
            pl.BlockSpec((1, 1, 1, K_CHUNK), lambda l, k: (l, k, 0, 0)),
            pl.BlockSpec((1, 1, DIM), lambda l, k: (l, 0, 0)),
            pl.BlockSpec((1, K_CHUNK, DIM), lambda l, k: (l, k, 0)),
        ],
        out_specs=pl.BlockSpec((1, 1, DIM), lambda l, k: (l, 0, 0)),
        out_shape=jax.ShapeDtypeStruct((NUM_LAYERS, 1, DIM), jnp.float32),
        compiler_params=pltpu.CompilerParams(
            dimension_semantics=("parallel", "arbitrary"),
        ),
    )(cs, bs3, bc3, event_counts, bs_k, bc_k, ev3, bank_evidence)
    return out.reshape(NUM_LAYERS, DIM)


# speedup vs baseline: 2.9546x; 1.8331x over previous
import jax
import jax.numpy as jnp
from jax.experimental import pallas as pl
from jax.experimental.pallas import tpu as pltpu

NUM_LAYERS = 32
DIM = 2048
TTL = 1024
HALF_LIFE = 256.0


def _body(cs_ref, bs_ref, bc_ref, ec_ref, ev_ref, bank_ref, out_ref):
    l = pl.program_id(0)
    cs = cs_ref[0]
    ec = ec_ref[l]

    bs = bs_ref[0]
    bc = bc_ref[0]
    iota = jax.lax.broadcasted_iota(jnp.int32, (1, TTL), 1)
    big = jnp.int32(TTL)
    empty = bs == -1
    first_empty = jnp.min(jnp.where(empty, iota, big))
    minstep = jnp.min(bs)
    oldest = jnp.min(jnp.where(bs == minstep, iota, big))
    slot = jnp.where(first_empty < big, first_empty, oldest)

    valid = (bs >= 0).astype(jnp.float32)
    age = jnp.maximum(cs - bs, 0).astype(jnp.float32)
    w = jnp.exp2(-age / HALF_LIFE) * valid * bc
    w = jnp.where(iota == slot, 0.0, w)
    wsum = jnp.sum(w) + ec

    acc = jnp.dot(w, bank_ref[0], preferred_element_type=jnp.float32)
    acc = acc + ec * ev_ref[0]
    res = acc / jnp.maximum(wsum, 1e-12)
    out_ref[0] = jnp.where(wsum > 0, res, jnp.zeros_like(res))


def kernel(evidence, event_counts, current_step, bank_evidence, bank_step,
           bank_event_count):
    cs = jnp.asarray(current_step, dtype=jnp.int32).reshape(1)
    bs3 = bank_step.reshape(NUM_LAYERS, 1, TTL)
    bc3 = bank_event_count.reshape(NUM_LAYERS, 1, TTL)
    ev3 = evidence.reshape(NUM_LAYERS, 1, DIM)

    out = pl.pallas_call(
        _body,
        grid=(NUM_LAYERS,),
        in_specs=[
            pl.BlockSpec(memory_space=pltpu.SMEM),
            pl.BlockSpec((1, 1, TTL), lambda l: (l, 0, 0)),
            pl.BlockSpec((1, 1, TTL), lambda l: (l, 0, 0)),
            pl.BlockSpec(memory_space=pltpu.SMEM),
            pl.BlockSpec((1, 1, DIM), lambda l: (l, 0, 0)),
            pl.BlockSpec((1, TTL, DIM), lambda l: (l, 0, 0)),
        ],
        out_specs=pl.BlockSpec((1, 1, DIM), lambda l: (l, 0, 0)),
        out_shape=jax.ShapeDtypeStruct((NUM_LAYERS, 1, DIM), jnp.float32),
        compiler_params=pltpu.CompilerParams(
            dimension_semantics=("arbitrary",),
        ),
    )(cs, bs3, bc3, event_counts, ev3, bank_evidence)
    return out.reshape(NUM_LAYERS, DIM)
